# baseline (speedup 1.0000x reference)
import functools

import jax
import jax.numpy as jnp
from jax import lax
from jax.experimental import pallas as pl
from jax.experimental.pallas import tpu as pltpu
from jax.experimental.pallas import tpu_sc as plsc

B, L, NL, NF, NV = 16, 128, 40, 32, 20
NW = 32
RPW = (B * NV) // NW
PROW = NF * NL
VB = 32

_mesh = plsc.VectorSubcoreMesh(core_axis_name="c", subcore_axis_name="s")


def _wmax_body(lp_ref, w_ref):
    x = lp_ref[0]
    xmax = jnp.max(x, axis=1)
    w = jnp.maximum(1.0 - jnp.exp(xmax), 1e-6)
    w_ref[0] = jnp.concatenate(
        [w, jnp.ones((VB, 128 - NL), jnp.float32)], axis=1)


def _pool_plan():
    plans = []
    for f in range(NF):
        a = f * NL
        sl = []
        for s, wo in ((a, 8), (a + 16, 24), (a + 24, 32)):
            if s % 128 == 120:
                sl.append((s - 8, wo - 8))
                sl.append((s + 8, wo + 8))
            else:
                sl.append((s, wo))
        plans.append(sl)
    return plans


_POOL_PLANS = _pool_plan()


@functools.partial(
    pl.kernel,
    out_type=jax.ShapeDtypeStruct((NW, RPW, 4, 128), jnp.float32),
    mesh=_mesh,
    scratch_types=[
        pltpu.VMEM((NW, RPW), jnp.int32),
        pltpu.VMEM((NW, RPW), jnp.int32),
        pltpu.VMEM((8, 128), jnp.float32),
        pltpu.VMEM((2, 128), jnp.float32),
        pltpu.VMEM((8, PROW), jnp.int32),
        pltpu.VMEM((2, PROW), jnp.int32),
        pltpu.VMEM((RPW, 4, 128), jnp.float32),
        pltpu.VMEM((64,), jnp.float32),
        pltpu.SemaphoreType.DMA,
        pltpu.SemaphoreType.DMA,
    ],
)
def _sc_poolmin(w_hbm, pool_hbm, ridx_hbm, fidx_hbm,
                wm_out,
                idx_all, fidx_all, w_a, w_b, pool_a, pool_b, wm_v, wbuf,
                sem0, sem1):
    wid = lax.axis_index("s") * 2 + lax.axis_index("c")
    pltpu.sync_copy(ridx_hbm, idx_all)
    pltpu.sync_copy(fidx_hbm, fidx_all)
    cp0 = pltpu.async_copy(w_hbm.at[idx_all.at[wid, pl.ds(0, 8)]],
                           w_a, sem0)
    cp1 = pltpu.async_copy(w_hbm.at[idx_all.at[wid, pl.ds(8, 2)]],
                           w_b, sem0)
    cp2 = pltpu.async_copy(pool_hbm.at[fidx_all.at[wid, pl.ds(0, 8)]],
                           pool_a, sem1)
    cp3 = pltpu.async_copy(pool_hbm.at[fidx_all.at[wid, pl.ds(8, 2)]],
                           pool_b, sem1)
    cp0.wait()
    cp1.wait()
    cp2.wait()
    cp3.wait()

    def make_row_body(w_v, pool_v, j0):
        def row_body(j, carry):
            ones = jnp.ones((16,), jnp.float32)
            wbuf[pl.ds(0, 16)] = ones
            wbuf[pl.ds(8, 16)] = w_v[j, pl.ds(0, 16)]
            wbuf[pl.ds(24, 16)] = w_v[j, pl.ds(16, 16)]
            wbuf[pl.ds(40, 16)] = w_v[j, pl.ds(32, 16)]
            wbuf[pl.ds(48, 16)] = ones
            for f in range(NF):
                c = None
                for s, wo in _POOL_PLANS[f]:
                    wk = wbuf[pl.ds(wo, 16)]
                    ck = jnp.where(pool_v[j, pl.ds(s, 16)] == 0, wk, 1.0)
                    c = ck if c is None else jnp.minimum(c, ck)
                wm_v[j0 + j, f // 8, pl.ds((f % 8) * 16, 16)] = c
            return carry
        return row_body

    lax.fori_loop(0, 8, make_row_body(w_a, pool_a, 0), 0)
    lax.fori_loop(0, 2, make_row_body(w_b, pool_b, 8), 0)
    pltpu.sync_copy(wm_v, wm_out.at[wid])


def _finish_body(wm_ref, lf_ref, vlab_ref, vl_ref, out_ref):
    def body(i, acc):
        oh = (lax.broadcasted_iota(jnp.int32, (NV, L), 1)
              == vlab_ref[i]).astype(jnp.float32)
        fp = jnp.dot(oh, lf_ref[i],
                     preferred_element_type=jnp.float32)
        wm_i = jnp.min(wm_ref[i], axis=2)
        t = jnp.maximum(fp - jnp.log(wm_i), 0.0)
        mask = lax.broadcasted_iota(jnp.int32, (NV, 1), 0) < vl_ref[i]
        return acc + jnp.sum(jnp.where(mask, t, 0.0))

    total = lax.fori_loop(0, B, body, jnp.float32(0.0))
    tot = lax.fori_loop(0, B, lambda i, a: a + vl_ref[i, 0], 0)
    norm = jnp.maximum(tot, 1).astype(jnp.float32)
    out_ref[...] = jnp.full((1, 1), total / norm, jnp.float32)


@jax.jit
def _frame_role_loss(log_pa, v_label, v_l, log_frame, frame_idx, frame_pool):
    vlab = v_label.astype(jnp.int32)
    ridx = (jnp.arange(B, dtype=jnp.int32)[:, None] * L + vlab)
    fidx = jnp.take_along_axis(frame_idx.astype(jnp.int32), vlab, axis=1)

    w_all = pl.pallas_call(
        _wmax_body,
        grid=(B, L // VB),
        in_specs=[pl.BlockSpec((1, VB, L, NL), lambda i, b: (i, b, 0, 0))],
        out_shape=jax.ShapeDtypeStruct((B, L, 128), jnp.float32),
        out_specs=pl.BlockSpec((1, VB, 128), lambda i, b: (i, b, 0)),
    )(log_pa)

    wm = _sc_poolmin(w_all.reshape(B * L, 128),
                     frame_pool.astype(jnp.int32).reshape(-1, PROW),
                     ridx.reshape(NW, RPW), fidx.reshape(NW, RPW))

    loss = pl.pallas_call(
        _finish_body,
        in_specs=[
            pl.BlockSpec((B, NV, NF, 16), lambda: (0, 0, 0, 0)),
            pl.BlockSpec((B, L, NF), lambda: (0, 0, 0)),
            pl.BlockSpec((B, NV, 1), lambda: (0, 0, 0)),
            pl.BlockSpec((B, 1), lambda: (0, 0)),
        ],
        out_shape=jax.ShapeDtypeStruct((1, 1), jnp.float32),
    )(wm.reshape(B, NV, NF, 16), log_frame, vlab.reshape(B, NV, 1),
      v_l.reshape(B, 1).astype(jnp.int32))
    return loss.reshape(())


def kernel(log_pa, score, v_label, v_l, role_label, roleset_id, log_frame,
           frame_idx, frame_pool):
    return _frame_role_loss(log_pa, v_label, v_l, log_frame, frame_idx,
                            frame_pool)


# speedup vs baseline: 1.2763x; 1.2763x over previous
import functools

import jax
import jax.numpy as jnp
from jax import lax
from jax.experimental import pallas as pl
from jax.experimental.pallas import tpu as pltpu
from jax.experimental.pallas import tpu_sc as plsc

B, L, NL, NF, NV = 16, 128, 40, 32, 20
NW = 32
RPW = (B * NV) // NW
ROW = L * NL
PROW = NF * NL
NEG = -3.0e38

_mesh = plsc.VectorSubcoreMesh(core_axis_name="c", subcore_axis_name="s")


def _pool_plan():
    plans = []
    for f in range(NF):
        a = f * NL
        sl = []
        for s, wo in ((a, 8), (a + 16, 24), (a + 24, 32)):
            if s % 128 == 120:
                sl.append((s - 8, wo - 8))
                sl.append((s + 8, wo + 8))
            else:
                sl.append((s, wo))
        plans.append(sl)
    return plans


_POOL_PLANS = _pool_plan()


@functools.partial(
    pl.kernel,
    out_type=jax.ShapeDtypeStruct((NW, RPW, 4, 128), jnp.float32),
    mesh=_mesh,
    scratch_types=[
        pltpu.VMEM((NW, RPW), jnp.int32),
        pltpu.VMEM((NW, RPW), jnp.int32),
        pltpu.VMEM((8, ROW), jnp.float32),
        pltpu.VMEM((2, ROW), jnp.float32),
        pltpu.VMEM((8, PROW), jnp.int32),
        pltpu.VMEM((2, PROW), jnp.int32),
        pltpu.VMEM((RPW, 4, 128), jnp.float32),
        pltpu.VMEM((80,), jnp.float32),
        pltpu.VMEM((64,), jnp.float32),
        pltpu.SemaphoreType.DMA,
        pltpu.SemaphoreType.DMA,
    ],
)
def _sc_gather_reduce(lp_hbm, pool_hbm, ridx_hbm, fidx_hbm,
                      wm_out,
                      idx_all, fidx_all, rows_a, rows_b, pool_a,
                      pool_b, wm_v, s80, wbuf, sem0, sem1):
    wid = lax.axis_index("s") * 2 + lax.axis_index("c")
    pltpu.sync_copy(ridx_hbm, idx_all)
    pltpu.sync_copy(fidx_hbm, fidx_all)
    cp0 = pltpu.async_copy(lp_hbm.at[idx_all.at[wid, pl.ds(0, 8)]],
                           rows_a, sem0)
    cp1 = pltpu.async_copy(lp_hbm.at[idx_all.at[wid, pl.ds(8, 2)]],
                           rows_b, sem0)
    cp2 = pltpu.async_copy(pool_hbm.at[fidx_all.at[wid, pl.ds(0, 8)]],
                           pool_a, sem1)
    cp3 = pltpu.async_copy(pool_hbm.at[fidx_all.at[wid, pl.ds(8, 2)]],
                           pool_b, sem1)
    cp0.wait()
    cp1.wait()
    cp2.wait()
    cp3.wait()

    def make_row_body(rows_v, pool_v, j0):
        def row_body(j, carry):
            def g_body(g, accs):
                base = pl.multiple_of(g * 80, 16)
                return tuple(
                    jnp.maximum(a, rows_v[j, pl.ds(base + 16 * p, 16)])
                    for p, a in enumerate(accs)
                )

            init = tuple(jnp.full((16,), NEG, jnp.float32) for _ in range(5))
            accs = lax.fori_loop(0, ROW // 80, g_body, init)
            for p in range(5):
                s80[pl.ds(16 * p, 16)] = accs[p]
            f0 = jnp.maximum(s80[pl.ds(0, 16)], s80[pl.ds(40, 16)])
            f1 = jnp.maximum(s80[pl.ds(16, 16)], s80[pl.ds(56, 16)])
            f2 = jnp.maximum(s80[pl.ds(24, 16)], s80[pl.ds(64, 16)])
            ones = jnp.ones((16,), jnp.float32)
            wbuf[pl.ds(0, 16)] = ones
            wbuf[pl.ds(48, 16)] = ones
            wbuf[pl.ds(8, 16)] = jnp.maximum(1.0 - jnp.exp(f0), 1e-6)
            wbuf[pl.ds(24, 16)] = jnp.maximum(1.0 - jnp.exp(f1), 1e-6)
            wbuf[pl.ds(32, 16)] = jnp.maximum(1.0 - jnp.exp(f2), 1e-6)
            for f in range(NF):
                c = None
                for s, wo in _POOL_PLANS[f]:
                    wk = wbuf[pl.ds(wo, 16)]
                    ck = jnp.where(pool_v[j, pl.ds(s, 16)] == 0, wk, 1.0)
                    c = ck if c is None else jnp.minimum(c, ck)
                wm_v[j0 + j, f // 8, pl.ds((f % 8) * 16, 16)] = c
            return carry
        return row_body

    lax.fori_loop(0, 8, make_row_body(rows_a, pool_a, 0), 0)
    lax.fori_loop(0, 2, make_row_body(rows_b, pool_b, 8), 0)
    pltpu.sync_copy(wm_v, wm_out.at[wid])


def _finish_body(wm_ref, lf_ref, vlab_ref, vl_ref, out_ref):
    def body(i, acc):
        oh = (lax.broadcasted_iota(jnp.int32, (NV, L), 1)
              == vlab_ref[i]).astype(jnp.float32)
        fp = jnp.dot(oh, lf_ref[i],
                     preferred_element_type=jnp.float32)
        wm_i = jnp.min(wm_ref[i], axis=2)
        t = jnp.maximum(fp - jnp.log(wm_i), 0.0)
        mask = lax.broadcasted_iota(jnp.int32, (NV, 1), 0) < vl_ref[i]
        return acc + jnp.sum(jnp.where(mask, t, 0.0))

    total = lax.fori_loop(0, B, body, jnp.float32(0.0))
    tot = lax.fori_loop(0, B, lambda i, a: a + vl_ref[i, 0], 0)
    norm = jnp.maximum(tot, 1).astype(jnp.float32)
    out_ref[...] = jnp.full((1, 1), total / norm, jnp.float32)


@jax.jit
def _frame_role_loss(log_pa, v_label, v_l, log_frame, frame_idx, frame_pool):
    lp_flat = log_pa.reshape(B * L, ROW)
    pool_flat = frame_pool.reshape(-1, PROW).astype(jnp.int32)
    vlab = v_label.astype(jnp.int32)
    ridx = (jnp.arange(B, dtype=jnp.int32)[:, None] * L + vlab)
    fidx = jnp.take_along_axis(frame_idx.astype(jnp.int32), vlab, axis=1)
    wm = _sc_gather_reduce(
        lp_flat, pool_flat,
        ridx.reshape(NW, RPW), fidx.reshape(NW, RPW))
    loss = pl.pallas_call(
        _finish_body,
        in_specs=[
            pl.BlockSpec((B, NV, NF, 16), lambda: (0, 0, 0, 0)),
            pl.BlockSpec((B, L, NF), lambda: (0, 0, 0)),
            pl.BlockSpec((B, NV, 1), lambda: (0, 0, 0)),
            pl.BlockSpec((B, 1), lambda: (0, 0)),
        ],
        out_shape=jax.ShapeDtypeStruct((1, 1), jnp.float32),
    )(wm.reshape(B, NV, NF, 16), log_frame, vlab.reshape(B, NV, 1),
      v_l.reshape(B, 1).astype(jnp.int32))
    return loss.reshape(())


def kernel(log_pa, score, v_label, v_l, role_label, roleset_id, log_frame,
           frame_idx, frame_pool):
    return _frame_role_loss(log_pa, v_label, v_l, log_frame, frame_idx,
                            frame_pool)
